# baseline (speedup 1.0000x reference)
import jax
import jax.numpy as jnp
from jax import lax
from jax.experimental import pallas as pl
from jax.experimental.pallas import tpu as pltpu
from jax.experimental.pallas import tpu_sc as plsc

NUM_EMBEDDINGS = 1000000
EMBEDDING_DIM = 32
BATCH = 4096
SEQ_LEN = 200
NUM_IDX = BATCH * SEQ_LEN

TBLK = 800
WINDOW = 128

_MESH = dict(core_axis_name="core", subcore_axis_name="subcore")


def _table_rowmajor(table_t):
    @pl.kernel(
        out_type=jax.ShapeDtypeStruct((NUM_EMBEDDINGS, EMBEDDING_DIM),
                                      table_t.dtype),
        mesh=plsc.VectorSubcoreMesh(**_MESH),
        compiler_params=pltpu.CompilerParams(
            use_tc_tiling_on_sc=False, needs_layout_passes=False
        ),
    )
    def body(x_hbm, o_hbm):
        def step(x_vmem, o_vmem):
            @pl.loop(0, EMBEDDING_DIM)
            def _(c):
                col = jnp.full((16,), c, dtype=jnp.int32)
                for k in range(TBLK // 16):
                    row = 16 * k + lax.iota(jnp.int32, 16)
                    vals = x_vmem[c, pl.ds(16 * k, 16)]
                    plsc.store_scatter(o_vmem, [row, col], vals)

        pltpu.emit_pipeline(
            step,
            grid=(NUM_EMBEDDINGS // TBLK,),
            in_specs=[
                pl.BlockSpec((EMBEDDING_DIM, TBLK), index_map=lambda i: (0, i))
            ],
            out_specs=[
                pl.BlockSpec((TBLK, EMBEDDING_DIM), index_map=lambda i: (i, 0))
            ],
            core_axis_name=("core", "subcore"),
            dimension_semantics=(pltpu.PARALLEL,),
        )(x_hbm, o_hbm)

    return body(table_t)


def _gather_slabs(table_lin, idx_flat):
    @pl.kernel(
        out_type=jax.ShapeDtypeStruct((SEQ_LEN, EMBEDDING_DIM, BATCH),
                                      table_lin.dtype),
        mesh=plsc.VectorSubcoreMesh(**_MESH),
        compiler_params=pltpu.CompilerParams(
            use_tc_tiling_on_sc=False, needs_layout_passes=False
        ),
    )
    def body(x_hbm, i_hbm, o_hbm):
        def step(i_vmem, o_vmem):
            def with_rows(rows_vmem):
                pltpu.sync_copy(x_hbm.at[i_vmem.at[0]], rows_vmem)

                @pl.loop(0, EMBEDDING_DIM)
                def _(c):
                    col = jnp.full((16,), c, dtype=jnp.int32)
                    for k in range(WINDOW // 16):
                        row = 16 * k + lax.iota(jnp.int32, 16)
                        vals = plsc.load_gather(rows_vmem, [row, col])
                        o_vmem[0, c, pl.ds(16 * k, 16)] = vals

            pl.run_scoped(
                with_rows,
                pltpu.VMEM((WINDOW, EMBEDDING_DIM), table_lin.dtype),
            )

        pltpu.emit_pipeline(
            step,
            grid=(NUM_IDX // WINDOW,),
            in_specs=[
                pl.BlockSpec((1, WINDOW), index_map=lambda i: (0, i))
            ],
            out_specs=[
                pl.BlockSpec(
                    (1, EMBEDDING_DIM, WINDOW),
                    index_map=lambda i: (i // (BATCH // WINDOW), 0,
                                         i % (BATCH // WINDOW)),
                )
            ],
            core_axis_name=("core", "subcore"),
            dimension_semantics=(pltpu.PARALLEL,),
        )(i_hbm, o_hbm)

    return body(table_lin, idx_flat)


def kernel(np_batch, table):
    table_lin = _table_rowmajor(table.T)
    idx_t = jnp.swapaxes(np_batch, 0, 1).astype(jnp.int32).reshape(1, NUM_IDX)
    slabs = _gather_slabs(table_lin, idx_t)
    return jnp.transpose(slabs, (2, 0, 1))


# speedup vs baseline: 4.1964x; 4.1964x over previous
import jax
import jax.numpy as jnp
from jax.experimental import pallas as pl
from jax.experimental.pallas import tpu as pltpu
from jax.experimental.pallas import tpu_sc as plsc

NUM_EMBEDDINGS = 1000000
EMBEDDING_DIM = 32
BATCH = 4096
SEQ_LEN = 200
NUM_IDX = BATCH * SEQ_LEN

WINDOW = 512


def _gather_fn(table, idx_flat):
    vector_mesh = plsc.VectorSubcoreMesh(
        core_axis_name="core", subcore_axis_name="subcore"
    )

    @pl.kernel(
        out_type=jax.ShapeDtypeStruct((SEQ_LEN, BATCH, EMBEDDING_DIM),
                                      table.dtype),
        mesh=vector_mesh,
        compiler_params=pltpu.CompilerParams(use_tc_tiling_on_sc=False),
    )
    def kernel_body(x_hbm, i_hbm, o_hbm):
        def body(i_vmem, o_vmem):
            pltpu.sync_copy(x_hbm.at[i_vmem.at[0]], o_vmem.at[0])

        pltpu.emit_pipeline(
            body,
            grid=(NUM_IDX // WINDOW,),
            in_specs=[pl.BlockSpec((1, WINDOW), index_map=lambda i: (0, i))],
            out_specs=[
                pl.BlockSpec(
                    (1, WINDOW, EMBEDDING_DIM),
                    index_map=lambda i: (i // (BATCH // WINDOW),
                                         i % (BATCH // WINDOW), 0),
                )
            ],
            core_axis_name=("core", "subcore"),
            dimension_semantics=(pltpu.PARALLEL,),
        )(i_hbm, o_hbm)

    return kernel_body(table, idx_flat)


def kernel(np_batch, table):
    idx_t = jnp.swapaxes(np_batch, 0, 1).astype(jnp.int32).reshape(1, NUM_IDX)
    out_t = _gather_fn(table, idx_t)
    return jnp.transpose(out_t, (1, 0, 2))
